# speedup vs baseline: 1.5036x; 1.5036x over previous
import functools
import math

import jax
import jax.numpy as jnp
from jax import lax
from jax.experimental import pallas as pl
from jax.experimental.pallas import tpu as pltpu
from jax.experimental.pallas import tpu_sc as plsc

D_MODEL = 512
SCALE = math.sqrt(float(D_MODEL))


@functools.cache
def _make_sc_embed(V, D, B):
    info = plsc.get_sparse_core_info()
    NC, NS, L = info.num_cores, info.num_subcores, info.num_lanes
    NW = NC * NS
    assert B % NW == 0
    b_per_w = B // NW
    CH = 64
    assert b_per_w % CH == 0
    NCHUNK = b_per_w // CH

    mesh = plsc.VectorSubcoreMesh(core_axis_name="c", subcore_axis_name="s")

    @functools.partial(
        pl.kernel,
        mesh=mesh,
        out_type=jax.ShapeDtypeStruct((B, D), jnp.float32),
        scratch_types=[
            pltpu.VMEM((b_per_w,), jnp.int32),
            pltpu.VMEM((CH, D), jnp.float32),
            pltpu.VMEM((CH, D), jnp.float32),
            pltpu.SemaphoreType.DMA,
            pltpu.SemaphoreType.DMA,
        ],
    )
    def k(idx_hbm, table_hbm, out_hbm, idx_v, buf0, buf1, sem0, sem1):
        wid = lax.axis_index("s") * NC + lax.axis_index("c")
        base = wid * b_per_w
        pltpu.sync_copy(idx_hbm.at[pl.ds(base, b_per_w)], idx_v)

        bufs = (buf0, buf1)
        sems = (sem0, sem1)
        handles = [None] * NCHUNK
        handles[0] = pltpu.async_copy(
            table_hbm.at[idx_v.at[pl.ds(0, CH)]], bufs[0], sems[0])
        for c in range(NCHUNK):
            if c + 1 < NCHUNK:
                handles[c + 1] = pltpu.async_copy(
                    table_hbm.at[idx_v.at[pl.ds((c + 1) * CH, CH)]],
                    bufs[(c + 1) % 2], sems[(c + 1) % 2])
            handles[c].wait()
            buf = bufs[c % 2]

            def body(r, carry, buf=buf):
                for j in range(D // L):
                    buf[r, pl.ds(j * L, L)] = buf[r, pl.ds(j * L, L)] * SCALE
                return carry

            lax.fori_loop(0, CH, body, 0)
            pltpu.sync_copy(buf, out_hbm.at[pl.ds(base + c * CH, CH)])

    return k


def kernel(x, lut):
    B = x.shape[0] * x.shape[1]
    V, D = lut.shape
    k = _make_sc_embed(V, D, B)
    out = k(x.reshape(B), lut)
    return out.reshape(x.shape[0], x.shape[1], D)
